# baseline (speedup 1.0000x reference)
import functools

import jax
import jax.numpy as jnp
from jax import lax
from jax.experimental import pallas as pl
from jax.experimental.pallas import tpu as pltpu
from jax.experimental.pallas import tpu_sc as plsc

HIDDEN = 64
VOCAB = 1000
NUM_EMB = 32
EPAD = 128
NC = 2
NS = 16
NW = NC * NS

BATCH = 1024
SEQ = 50
VBLK = 32
LANES = 16


def _fused_mm_body(w_ref, emb_ref, out_ref):
    out_ref[...] = lax.dot_general(
        w_ref[...], emb_ref[...],
        dimension_numbers=(((1,), (1,)), ((), ())),
        preferred_element_type=jnp.float32,
    )


def _fused_table_t(emb_pad, W):
    return pl.pallas_call(
        _fused_mm_body,
        out_shape=jax.ShapeDtypeStruct((VOCAB, EPAD), jnp.float32),
    )(W, emb_pad)


def _sc_tgather_body(
    ft_hbm, ids_hbm, out_hbm, ft_v, ids_v, buf_a, buf_b, wsem_a, wsem_b
):
    wid = lax.axis_index("s") * NC + lax.axis_index("c")
    v0 = jnp.minimum(wid * VBLK, VOCAB - VBLK)

    def out_desc(l, buf, sem):
        return pltpu.make_async_copy(
            buf, out_hbm.at[l, pl.ds(v0, VBLK)], sem
        )

    pltpu.sync_copy(ids_hbm, ids_v)
    pltpu.sync_copy(ft_hbm.at[pl.ds(v0 * EPAD, VBLK * EPAD)], ft_v)

    def compute(l, buf_ref):
        base = l * BATCH

        def lgroup(i, c):
            off = i * LANES
            ids16 = ids_v[pl.ds(base + off, LANES)]
            vals = [
                plsc.load_gather(ft_v, [ids16 + vv * EPAD])
                for vv in range(VBLK)
            ]
            for vv in range(VBLK):
                buf_ref[vv, pl.ds(off, LANES)] = vals[vv]
            return c

        lax.fori_loop(0, BATCH // LANES, lgroup, 0)

    def fence(buf_ref):
        acc = buf_ref[0, pl.ds(BATCH - LANES, LANES)]
        for vv in range(1, VBLK):
            acc = acc + buf_ref[vv, pl.ds(BATCH - LANES, LANES)]
        for vv in range(VBLK):
            acc = acc + buf_ref[vv, pl.ds(BATCH - 2 * LANES, LANES)]
        s = jnp.sum(acc)
        return s == s

    def body(l, carry):
        compute(l, buf_a)

        @pl.when(fence(buf_a))
        def _():
            out_desc(l, buf_a, wsem_a).start()

        out_desc(l, buf_a, wsem_a).wait()
        return carry

    lax.fori_loop(0, SEQ, body, 0)


_sc_tgather = functools.partial(
    pl.kernel,
    out_type=jax.ShapeDtypeStruct((SEQ, VOCAB, BATCH), jnp.float32),
    mesh=plsc.VectorSubcoreMesh(core_axis_name="c", subcore_axis_name="s"),
    scratch_types=[
        pltpu.VMEM((VBLK * EPAD,), jnp.float32),
        pltpu.VMEM((SEQ * BATCH,), jnp.int32),
        pltpu.VMEM((VBLK, BATCH), jnp.float32),
        pltpu.VMEM((VBLK, BATCH), jnp.float32),
        pltpu.SemaphoreType.DMA,
        pltpu.SemaphoreType.DMA,
    ],
    compiler_params=pltpu.CompilerParams(
        use_tc_tiling_on_sc=True, needs_layout_passes=False
    ),
)(_sc_tgather_body)


def kernel(input_ids, emb_table, W):
    emb_pad = jnp.pad(emb_table, ((0, EPAD - NUM_EMB), (0, 0)))
    fused_t = _fused_table_t(emb_pad, W)
    ids_t = jnp.swapaxes(input_ids.astype(jnp.int32), 0, 1)
    out_t = _sc_tgather(fused_t.reshape(-1), ids_t.reshape(-1))
    return jnp.transpose(out_t, (2, 0, 1))


# speedup vs baseline: 1.3440x; 1.3440x over previous
import functools

import jax
import jax.numpy as jnp
from jax import lax
from jax.experimental import pallas as pl
from jax.experimental.pallas import tpu as pltpu
from jax.experimental.pallas import tpu_sc as plsc

HIDDEN = 64
VOCAB = 1000
NUM_EMB = 32
EPAD = 128
NC = 2
NS = 16
NW = NC * NS

BATCH = 1024
SEQ = 50
VBLK = 32
LANES = 16


def _fused_mm_body(w_ref, emb_ref, out_ref):
    out_ref[...] = lax.dot_general(
        w_ref[...], emb_ref[...],
        dimension_numbers=(((1,), (1,)), ((), ())),
        preferred_element_type=jnp.float32,
    )


def _fused_table_t(emb_pad, W):
    return pl.pallas_call(
        _fused_mm_body,
        out_shape=jax.ShapeDtypeStruct((VOCAB, EPAD), jnp.float32),
    )(W, emb_pad)


def _sc_tgather_body(
    ft_hbm, ids_hbm, out_hbm, ft_v, ids_v, buf_a, buf_b, wsem_a, wsem_b
):
    wid = lax.axis_index("s") * NC + lax.axis_index("c")
    v0 = jnp.minimum(wid * VBLK, VOCAB - VBLK)

    def out_desc(l, buf, sem):
        return pltpu.make_async_copy(
            buf, out_hbm.at[l, pl.ds(v0, VBLK)], sem
        )

    pltpu.sync_copy(ids_hbm, ids_v)
    pltpu.sync_copy(ft_hbm.at[pl.ds(v0 * EPAD, VBLK * EPAD)], ft_v)

    def compute(l, buf_ref):
        base = l * BATCH

        def lgroup(i, c):
            off = i * LANES
            ids16 = ids_v[pl.ds(base + off, LANES)]
            vals = [
                plsc.load_gather(ft_v, [ids16 + vv * EPAD])
                for vv in range(VBLK)
            ]
            for vv in range(VBLK):
                buf_ref[vv, pl.ds(off, LANES)] = vals[vv]
            return c

        lax.fori_loop(0, BATCH // LANES, lgroup, 0)

    def fence(buf_ref):
        acc = buf_ref[0, pl.ds(BATCH - LANES, LANES)]
        for vv in range(1, VBLK):
            acc = acc + buf_ref[vv, pl.ds(BATCH - LANES, LANES)]
        for vv in range(VBLK):
            acc = acc + buf_ref[vv, pl.ds(BATCH - 2 * LANES, LANES)]
        s = jnp.sum(acc)
        return s == s

    def body(j, carry):
        l0 = 2 * j
        l1 = l0 + 1

        @pl.when(j > 0)
        def _():
            out_desc(l0 - 2, buf_a, wsem_a).wait()

        compute(l0, buf_a)

        @pl.when(fence(buf_a))
        def _():
            out_desc(l0, buf_a, wsem_a).start()

        @pl.when(j > 0)
        def _():
            out_desc(l1 - 2, buf_b, wsem_b).wait()

        compute(l1, buf_b)

        @pl.when(fence(buf_b))
        def _():
            out_desc(l1, buf_b, wsem_b).start()

        return carry

    lax.fori_loop(0, SEQ // 2, body, 0)
    out_desc(SEQ - 2, buf_a, wsem_a).wait()
    out_desc(SEQ - 1, buf_b, wsem_b).wait()


_sc_tgather = functools.partial(
    pl.kernel,
    out_type=jax.ShapeDtypeStruct((SEQ, VOCAB, BATCH), jnp.float32),
    mesh=plsc.VectorSubcoreMesh(core_axis_name="c", subcore_axis_name="s"),
    scratch_types=[
        pltpu.VMEM((VBLK * EPAD,), jnp.float32),
        pltpu.VMEM((SEQ * BATCH,), jnp.int32),
        pltpu.VMEM((VBLK, BATCH), jnp.float32),
        pltpu.VMEM((VBLK, BATCH), jnp.float32),
        pltpu.SemaphoreType.DMA,
        pltpu.SemaphoreType.DMA,
    ],
    compiler_params=pltpu.CompilerParams(
        use_tc_tiling_on_sc=True, needs_layout_passes=False
    ),
)(_sc_tgather_body)


def kernel(input_ids, emb_table, W):
    emb_pad = jnp.pad(emb_table, ((0, EPAD - NUM_EMB), (0, 0)))
    fused_t = _fused_table_t(emb_pad, W)
    ids_t = jnp.swapaxes(input_ids.astype(jnp.int32), 0, 1)
    out_t = _sc_tgather(fused_t.reshape(-1), ids_t.reshape(-1))
    return jnp.transpose(out_t, (2, 0, 1))
